# baseline (speedup 1.0000x reference)
import jax
import jax.numpy as jnp
from jax import lax
from jax.experimental import pallas as pl
from jax.experimental.pallas import tpu as pltpu
from jax.experimental.pallas import tpu_sc as plsc

L = 16
C_DIM = 96
N_WORKERS = 32
CHUNK = 56


def _unpool_body(x_hbm, ind_hbm, out_hbm,
                 ind_c, x_c, out_a, out_b,
                 s_i, s_x, s_oa, s_ob):
    n_pairs, h, w = ind_hbm.shape
    oh, ow = out_a.shape
    t_stride = C_DIM
    b_stride = 2 * C_DIM

    wid = lax.axis_index("s") * 2 + lax.axis_index("c")
    pairs_per_w = n_pairs // N_WORKERS
    n_planes = 2 * pairs_per_w

    o_bufs = (out_a, out_b)
    o_sems = (s_oa, s_ob)

    zeros16 = jnp.zeros((L,), jnp.float32)

    def memset_plane(ref):
        def body(r, _):
            for u in range(ow // L):
                ref[r, pl.ds(u * L, L)] = zeros16
            return 0

        with jax.named_scope("memset"):
            lax.fori_loop(0, oh, body, 0)

    def scatter_chunk(ref):
        def body(r, _):
            for u in range(w // L):
                idx = ind_c[r, pl.ds(u * L, L)]
                val = x_c[r, pl.ds(u * L, L)]
                q5 = jax.lax.shift_right_logical(idx, 5)
                row = jax.lax.shift_right_logical(q5 * 9363, 16)
                col = idx - row * ow
                plsc.store_scatter(ref, [row, col], val)
            return 0

        with jax.named_scope("scatter"):
            lax.fori_loop(0, CHUNK, body, 0)

    def plane_refs(q):
        k, t = q // 2, q % 2
        p = wid * pairs_per_w + k
        b = p // C_DIM
        c = p - b * C_DIM
        r = b * b_stride + t * t_stride + c
        return p, r

    def issue_chunk(q, c):
        p, r = plane_refs(q)
        hi = pltpu.async_copy(
            ind_hbm.at[p, pl.ds(c * CHUNK, CHUNK)], ind_c, s_i)
        hx = pltpu.async_copy(
            x_hbm.at[r, pl.ds(c * CHUNK, CHUNK)], x_c, s_x)
        return hi, hx

    pending = issue_chunk(0, 0)
    memset_plane(out_a)
    memset_plane(out_b)
    h_out = [None, None]

    for q in range(n_planes):
        obuf = o_bufs[q % 2]
        _, r = plane_refs(q)
        if h_out[q % 2] is not None:
            h_out[q % 2].wait()
            memset_plane(obuf)
        pending[0].wait()
        pending[1].wait()
        scatter_chunk(obuf)
        pending = issue_chunk(q, 1)
        pending[0].wait()
        pending[1].wait()
        scatter_chunk(obuf)
        if q + 1 < n_planes:
            pending = issue_chunk(q + 1, 0)
        h_out[q % 2] = pltpu.async_copy(obuf, out_hbm.at[r], o_sems[q % 2])

    h_out[0].wait()
    h_out[1].wait()


@jax.jit
def kernel(x, ind):
    bb, tt, cc, h, ww = x.shape
    x3 = x.reshape(bb * tt * cc, h, ww)
    ind3 = ind.reshape(bb * cc, h, ww)
    mesh = plsc.VectorSubcoreMesh(
        core_axis_name="c", subcore_axis_name="s", num_cores=2, num_subcores=16
    )
    run = pl.kernel(
        _unpool_body,
        out_type=jax.ShapeDtypeStruct((bb * tt * cc, 2 * h, 2 * ww), jnp.float32),
        mesh=mesh,
        scratch_types=[
            pltpu.VMEM((CHUNK, ww), jnp.int32),
            pltpu.VMEM((CHUNK, ww), jnp.float32),
            pltpu.VMEM((2 * h, 2 * ww), jnp.float32),
            pltpu.VMEM((2 * h, 2 * ww), jnp.float32),
            pltpu.SemaphoreType.DMA,
            pltpu.SemaphoreType.DMA,
            pltpu.SemaphoreType.DMA,
            pltpu.SemaphoreType.DMA,
        ],
        compiler_params=pltpu.CompilerParams(needs_layout_passes=False),
    )
    out = run(x3, ind3)
    return out.reshape(bb, tt, cc, 2 * h, 2 * ww)


# speedup vs baseline: 1.7040x; 1.7040x over previous
import jax
import jax.numpy as jnp
from jax import lax
from jax.experimental import pallas as pl
from jax.experimental.pallas import tpu as pltpu
from jax.experimental.pallas import tpu_sc as plsc

L = 16
C_DIM = 96
N_WORKERS = 32
CHUNK = 56


def _unpool_body(x_hbm, ind_hbm, out_hbm,
                 ind_c, x_c, out_a, out_b,
                 s_i, s_x, s_oa, s_ob):
    n_pairs, h, w = ind_hbm.shape
    oh, ow = out_a.shape

    t_stride = C_DIM
    b_stride = 2 * C_DIM

    wid = lax.axis_index("s") * 2 + lax.axis_index("c")
    pairs_per_w = n_pairs // N_WORKERS
    n_planes = 2 * pairs_per_w

    o_bufs = (out_a, out_b)
    o_sems = (s_oa, s_ob)

    zeros16 = jnp.zeros((L,), jnp.float32)

    def memset_plane(ref):
        def body(r, _):
            for u in range(ow // L):
                ref[r, pl.ds(u * L, L)] = zeros16
            return 0

        with jax.named_scope("memset"):
            lax.fori_loop(0, oh, body, 0)

    def compute_offsets():
        def body(r):
            for u in range(w // L):
                idx = ind_c[r, pl.ds(u * L, L)]
                q5 = jax.lax.shift_right_logical(idx, 5)
                row = jax.lax.shift_right_logical(q5 * 9363, 16)
                col = idx - row * ow
                ind_c[r, pl.ds(u * L, L)] = (
                    jax.lax.shift_left(row, 16) + col
                )

        with jax.named_scope("offsets"):
            plsc.parallel_loop(0, CHUNK, 1, unroll=2)(body)

    def scatter_chunk(ref):
        nv = w // L

        def body(r, _):
            packed = [ind_c[r, pl.ds(u * L, L)] for u in range(nv)]
            vals = [x_c[r, pl.ds(u * L, L)] for u in range(nv)]
            for u in range(nv):
                row = jax.lax.shift_right_logical(packed[u], 16)
                col = jnp.bitwise_and(packed[u], 65535)
                plsc.store_scatter(ref, [row, col], vals[u])
            return 0

        with jax.named_scope("scatter"):
            lax.fori_loop(0, CHUNK, body, 0)

    def plane_refs(q):
        k, t = q // 2, q % 2
        p = wid * pairs_per_w + k
        b = p // C_DIM
        c = p - b * C_DIM
        r = b * b_stride + t * t_stride + c
        return p, r

    def issue_chunk(q, c):
        p, r = plane_refs(q)
        hi = pltpu.async_copy(
            ind_hbm.at[p, pl.ds(c * CHUNK, CHUNK)], ind_c, s_i)
        hx = pltpu.async_copy(
            x_hbm.at[r, pl.ds(c * CHUNK, CHUNK)], x_c, s_x)
        return hi, hx

    pending = issue_chunk(0, 0)
    memset_plane(out_a)
    memset_plane(out_b)
    h_out = [None, None]

    for q in range(n_planes):
        obuf = o_bufs[q % 2]
        _, r = plane_refs(q)
        if h_out[q % 2] is not None:
            h_out[q % 2].wait()
            memset_plane(obuf)
        pending[0].wait()
        pending[1].wait()
        compute_offsets()
        scatter_chunk(obuf)
        pending = issue_chunk(q, 1)
        pending[0].wait()
        pending[1].wait()
        compute_offsets()
        scatter_chunk(obuf)
        if q + 1 < n_planes:
            pending = issue_chunk(q + 1, 0)
        h_out[q % 2] = pltpu.async_copy(obuf, out_hbm.at[r], o_sems[q % 2])

    h_out[0].wait()
    h_out[1].wait()


@jax.jit
def kernel(x, ind):
    bb, tt, cc, h, ww = x.shape
    x3 = x.reshape(bb * tt * cc, h, ww)
    ind3 = ind.reshape(bb * cc, h, ww)
    mesh = plsc.VectorSubcoreMesh(
        core_axis_name="c", subcore_axis_name="s", num_cores=2, num_subcores=16
    )
    run = pl.kernel(
        _unpool_body,
        out_type=jax.ShapeDtypeStruct((bb * tt * cc, 2 * h, 2 * ww), jnp.float32),
        mesh=mesh,
        scratch_types=[
            pltpu.VMEM((CHUNK, ww), jnp.int32),
            pltpu.VMEM((CHUNK, ww), jnp.float32),
            pltpu.VMEM((2 * h, 2 * ww), jnp.float32),
            pltpu.VMEM((2 * h, 2 * ww), jnp.float32),
            pltpu.SemaphoreType.DMA,
            pltpu.SemaphoreType.DMA,
            pltpu.SemaphoreType.DMA,
            pltpu.SemaphoreType.DMA,
        ],
        compiler_params=pltpu.CompilerParams(needs_layout_passes=False),
    )
    out = run(x3, ind3)
    return out.reshape(bb, tt, cc, 2 * h, 2 * ww)


# speedup vs baseline: 1.7072x; 1.0019x over previous
import jax
import jax.numpy as jnp
from jax import lax
from jax.experimental import pallas as pl
from jax.experimental.pallas import tpu as pltpu
from jax.experimental.pallas import tpu_sc as plsc

L = 16
C_DIM = 96
N_WORKERS = 32
CHUNK = 56


def _unpool_body(x_hbm, ind_hbm, out_hbm,
                 ind_c, x_c, out_a, out_b,
                 s_i, s_x, s_oa, s_ob):
    n_pairs, h, w = ind_hbm.shape
    oh, ow = out_a.shape

    t_stride = C_DIM
    b_stride = 2 * C_DIM

    wid = lax.axis_index("s") * 2 + lax.axis_index("c")
    pairs_per_w = n_pairs // N_WORKERS
    n_planes = 2 * pairs_per_w

    o_bufs = (out_a, out_b)
    o_sems = (s_oa, s_ob)

    zeros16 = jnp.zeros((L,), jnp.float32)

    def memset_plane(ref):
        def body(r, _):
            for u in range(ow // L):
                ref[r, pl.ds(u * L, L)] = zeros16
            return 0

        lax.fori_loop(0, oh, body, 0)

    def compute_offsets():
        def body(r):
            for u in range(w // L):
                idx = ind_c[r, pl.ds(u * L, L)]
                q5 = jax.lax.shift_right_logical(idx, 5)
                row = jax.lax.shift_right_logical(q5 * 9363, 16)
                col = idx - row * ow
                ind_c[r, pl.ds(u * L, L)] = (
                    jax.lax.shift_left(row, 16) + col
                )

        plsc.parallel_loop(0, CHUNK, 1, unroll=2)(body)

    def scatter_chunk(ref):
        nv = w // L

        def body(r, _):
            packed = [ind_c[r, pl.ds(u * L, L)] for u in range(nv)]
            vals = [x_c[r, pl.ds(u * L, L)] for u in range(nv)]
            for u in range(nv):
                row = jax.lax.shift_right_logical(packed[u], 16)
                col = jnp.bitwise_and(packed[u], 65535)
                plsc.store_scatter(ref, [row, col], vals[u])
            return 0

        lax.fori_loop(0, CHUNK, body, 0)

    def plane_refs(q):
        k, t = q // 2, q % 2
        p = wid * pairs_per_w + k
        b = p // C_DIM
        c = p - b * C_DIM
        r = b * b_stride + t * t_stride + c
        return p, r

    def issue_chunk(q, c):
        p, r = plane_refs(q)
        hi = pltpu.async_copy(
            ind_hbm.at[p, pl.ds(c * CHUNK, CHUNK)], ind_c, s_i)
        hx = pltpu.async_copy(
            x_hbm.at[r, pl.ds(c * CHUNK, CHUNK)], x_c, s_x)
        return hi, hx

    pending = issue_chunk(0, 0)
    memset_plane(out_a)
    memset_plane(out_b)
    h_out = [None, None]

    for q in range(n_planes):
        obuf = o_bufs[q % 2]
        _, r = plane_refs(q)
        if h_out[q % 2] is not None:
            h_out[q % 2].wait()
            memset_plane(obuf)
        pending[0].wait()
        pending[1].wait()
        compute_offsets()
        scatter_chunk(obuf)
        pending = issue_chunk(q, 1)
        pending[0].wait()
        pending[1].wait()
        compute_offsets()
        scatter_chunk(obuf)
        if q + 1 < n_planes:
            pending = issue_chunk(q + 1, 0)
        h_out[q % 2] = pltpu.async_copy(obuf, out_hbm.at[r], o_sems[q % 2])

    h_out[0].wait()
    h_out[1].wait()


@jax.jit
def kernel(x, ind):
    bb, tt, cc, h, ww = x.shape
    x3 = x.reshape(bb * tt * cc, h, ww)
    ind3 = ind.reshape(bb * cc, h, ww)
    mesh = plsc.VectorSubcoreMesh(
        core_axis_name="c", subcore_axis_name="s", num_cores=2, num_subcores=16
    )
    run = pl.kernel(
        _unpool_body,
        out_type=jax.ShapeDtypeStruct((bb * tt * cc, 2 * h, 2 * ww), jnp.float32),
        mesh=mesh,
        scratch_types=[
            pltpu.VMEM((CHUNK, ww), jnp.int32),
            pltpu.VMEM((CHUNK, ww), jnp.float32),
            pltpu.VMEM((2 * h, 2 * ww), jnp.float32),
            pltpu.VMEM((2 * h, 2 * ww), jnp.float32),
            pltpu.SemaphoreType.DMA,
            pltpu.SemaphoreType.DMA,
            pltpu.SemaphoreType.DMA,
            pltpu.SemaphoreType.DMA,
        ],
        compiler_params=pltpu.CompilerParams(needs_layout_passes=False),
    )
    out = run(x3, ind3)
    return out.reshape(bb, tt, cc, 2 * h, 2 * ww)
